# Initial kernel scaffold; baseline (speedup 1.0000x reference)
import jax
import jax.numpy as jnp
from jax.experimental import pallas as pl


def kernel(latents, emb_weight):
    raise NotImplementedError("write your pallas kernel here")


# speedup vs baseline: 1.2456x; 1.2456x over previous
import functools

import jax
import jax.numpy as jnp
from jax import lax
from jax.experimental import pallas as pl
from jax.experimental.pallas import tpu as pltpu
from jax.experimental.pallas import tpu_sc as plsc

_K = 8192
_D = 32
_B = 16
_N = 1024
_KT = 1024


_CHUNKS = ((0, 2816), (2816, 2816), (5632, 2560))
_MT = 256


def _nearest_code_kernel(w_ref, x_ref, x2_ref, w2_ref, idx_ref):
    x = x_ref[0]
    x2 = x2_ref[0]
    x_bf = x.astype(jnp.bfloat16)
    run_min = jnp.full((1, _N), jnp.inf, dtype=jnp.float32)
    run_arg = jnp.zeros((1, _N), dtype=jnp.int32)
    for start, size in _CHUNKS:
        cm = jnp.full((1, _N), jnp.inf, dtype=jnp.float32)
        ci = jnp.zeros((1, _N), dtype=jnp.int32)
        for t in range(size // _MT):
            s = start + t * _MT
            wt = w_ref[s:s + _MT, :].astype(jnp.bfloat16)
            w2t = w2_ref[s:s + _MT, :]
            cross = jnp.dot(wt, x_bf,
                            preferred_element_type=jnp.float32)
            d2 = (x2 + w2t) - 2.0 * cross
            dist = jnp.sqrt(jnp.maximum(d2, 0.0))
            tmin = jnp.min(dist, axis=0, keepdims=True)
            kio = lax.broadcasted_iota(jnp.int32, (_MT, _N), 0) + s
            targ = jnp.min(jnp.where(dist == tmin, kio, jnp.int32(2**30)),
                           axis=0, keepdims=True)
            better = tmin < cm
            ci = jnp.where(better, targ, ci)
            cm = jnp.where(better, tmin, cm)
        take = cm < run_min
        run_arg = jnp.where(take, ci, run_arg)
        run_min = jnp.where(take, cm, run_min)
        run_min = run_min.astype(jnp.bfloat16).astype(jnp.float32)
    idx_ref[0] = run_arg


def _nearest_codes(emb_weight, latents, x2, w2):
    return pl.pallas_call(
        _nearest_code_kernel,
        grid=(_B,),
        in_specs=[
            pl.BlockSpec((_K, _D), lambda b: (0, 0)),
            pl.BlockSpec((1, _D, _N), lambda b: (b, 0, 0)),
            pl.BlockSpec((1, 1, _N), lambda b: (b, 0, 0)),
            pl.BlockSpec((_K, 1), lambda b: (0, 0)),
        ],
        out_specs=pl.BlockSpec((1, 1, _N), lambda b: (b, 0, 0)),
        out_shape=jax.ShapeDtypeStruct((_B, 1, _N), jnp.int32),
    )(emb_weight, latents, x2, w2)


_NC = 2
_NS = 16
_NW = _NC * _NS
_BN = _B * _N
_PER_W = _BN // _NW


@functools.cache
def _sc_gather_fn():
    @functools.partial(
        pl.kernel,
        out_type=jax.ShapeDtypeStruct((_BN, _D), jnp.float32),
        mesh=plsc.VectorSubcoreMesh(core_axis_name="c", subcore_axis_name="s"),
        scratch_types=[
            pltpu.VMEM((_PER_W,), jnp.int32),
            pltpu.VMEM((_PER_W, _D), jnp.float32),
            pltpu.SemaphoreType.DMA,
        ],
        compiler_params=pltpu.CompilerParams(use_tc_tiling_on_sc=False),
    )
    def _sc_gather(table_hbm, idx_hbm, out_hbm, idx_v, rows_v, sem):
        wid = lax.axis_index("s") * _NC + lax.axis_index("c")
        base = wid * _PER_W
        pltpu.sync_copy(idx_hbm.at[pl.ds(base, _PER_W)], idx_v)
        pltpu.async_copy(table_hbm.at[idx_v], rows_v, sem).wait()
        pltpu.sync_copy(rows_v, out_hbm.at[pl.ds(base, _PER_W)])

    return _sc_gather


def kernel(latents, emb_weight):
    x = jnp.transpose(latents, (0, 2, 1))
    x2 = jnp.sum(x * x, axis=-1).reshape(_B, 1, _N)
    w2 = jnp.sum(emb_weight * emb_weight, axis=-1).reshape(_K, 1)
    idx = _nearest_codes(emb_weight, latents, x2, w2)
    rows = _sc_gather_fn()(emb_weight, idx.reshape(_BN))
    return jnp.transpose(rows.reshape(_B, _N, _D), (0, 2, 1))


# speedup vs baseline: 1.3267x; 1.0651x over previous
import functools

import jax
import jax.numpy as jnp
from jax import lax
from jax.experimental import pallas as pl
from jax.experimental.pallas import tpu as pltpu
from jax.experimental.pallas import tpu_sc as plsc

_K = 8192
_D = 32
_B = 16
_N = 1024
_KT = 1024


_CHUNKS = ((0, 2816), (2816, 2816), (5632, 2560))
_MT = 512


def _tiles(start, size):
    s = start
    while s < start + size:
        mt = min(_MT, start + size - s)
        yield s, mt
        s += mt


def _nearest_code_kernel(w2x_ref, x_ref, x2_ref, w2_ref, idx_ref):
    x = x_ref[0]
    x2 = x2_ref[0]
    x_bf = x.astype(jnp.bfloat16)
    run_min = jnp.full((1, _N), jnp.inf, dtype=jnp.float32)
    run_arg = jnp.zeros((1, _N), dtype=jnp.int32)
    for start, size in _CHUNKS:
        cm = jnp.full((1, _N), jnp.inf, dtype=jnp.float32)
        ci = jnp.zeros((1, _N), dtype=jnp.int32)
        for s, mt in _tiles(start, size):
            wt = w2x_ref[s:s + mt, :].astype(jnp.bfloat16)
            w2t = w2_ref[s:s + mt, :]
            cross2 = jnp.dot(wt, x_bf,
                             preferred_element_type=jnp.float32)
            d2 = (x2 + w2t) - cross2
            dist = jnp.sqrt(jnp.maximum(d2, 0.0))
            tmin = jnp.min(dist, axis=0, keepdims=True)
            kio = lax.broadcasted_iota(jnp.int32, (mt, _N), 0) + s
            targ = jnp.min(jnp.where(dist == tmin, kio, jnp.int32(2**30)),
                           axis=0, keepdims=True)
            better = tmin < cm
            ci = jnp.where(better, targ, ci)
            cm = jnp.where(better, tmin, cm)
        take = cm < run_min
        run_arg = jnp.where(take, ci, run_arg)
        run_min = jnp.where(take, cm, run_min)
        run_min = run_min.astype(jnp.bfloat16).astype(jnp.float32)
    idx_ref[0] = run_arg


def _nearest_codes(emb_weight, latents, x2, w2):
    return pl.pallas_call(
        _nearest_code_kernel,
        grid=(_B,),
        in_specs=[
            pl.BlockSpec((_K, _D), lambda b: (0, 0)),
            pl.BlockSpec((1, _D, _N), lambda b: (b, 0, 0)),
            pl.BlockSpec((1, 1, _N), lambda b: (b, 0, 0)),
            pl.BlockSpec((_K, 1), lambda b: (0, 0)),
        ],
        out_specs=pl.BlockSpec((1, 1, _N), lambda b: (b, 0, 0)),
        out_shape=jax.ShapeDtypeStruct((_B, 1, _N), jnp.int32),
    )(emb_weight, latents, x2, w2)


_NC = 2
_NS = 16
_NW = _NC * _NS
_BN = _B * _N
_PER_W = _BN // _NW


@functools.cache
def _sc_gather_fn():
    @functools.partial(
        pl.kernel,
        out_type=jax.ShapeDtypeStruct((_BN, _D), jnp.float32),
        mesh=plsc.VectorSubcoreMesh(core_axis_name="c", subcore_axis_name="s"),
        scratch_types=[
            pltpu.VMEM((_PER_W,), jnp.int32),
            pltpu.VMEM((_PER_W, _D), jnp.float32),
            pltpu.SemaphoreType.DMA,
        ],
        compiler_params=pltpu.CompilerParams(use_tc_tiling_on_sc=False),
    )
    def _sc_gather(table_hbm, idx_hbm, out_hbm, idx_v, rows_v, sem):
        wid = lax.axis_index("s") * _NC + lax.axis_index("c")
        base = wid * _PER_W
        pltpu.sync_copy(idx_hbm.at[pl.ds(base, _PER_W)], idx_v)
        pltpu.async_copy(table_hbm.at[idx_v], rows_v, sem).wait()
        pltpu.sync_copy(rows_v, out_hbm.at[pl.ds(base, _PER_W)])

    return _sc_gather


def kernel(latents, emb_weight):
    x = jnp.transpose(latents, (0, 2, 1))
    x2 = jnp.sum(x * x, axis=-1).reshape(_B, 1, _N)
    w2 = jnp.sum(emb_weight * emb_weight, axis=-1).reshape(_K, 1)
    idx = _nearest_codes(2.0 * emb_weight, latents, x2, w2)
    rows = _sc_gather_fn()(emb_weight, idx.reshape(_BN))
    return jnp.transpose(rows.reshape(_B, _N, _D), (0, 2, 1))
